# Initial kernel scaffold; baseline (speedup 1.0000x reference)
import jax
import jax.numpy as jnp
from jax.experimental import pallas as pl


def kernel(x, edge_index, edge_attr, W_edge, b_edge, W_gat, att_src, att_dst, bias_gat):
    raise NotImplementedError("write your pallas kernel here")


# speedup vs baseline: 40.9477x; 40.9477x over previous
import functools

import jax
import jax.numpy as jnp
from jax import lax
from jax.experimental import pallas as pl
from jax.experimental.pallas import tpu as pltpu
from jax.experimental.pallas import tpu_sc as plsc

N = 10000
E = 320000
ED = 16
ND = 128

NC = 2
NS = 16
NW = NC * NS
EPT = E // NW
CHUNK = 2000
NCHUNKS = EPT // CHUNK
SL = 640
NPAD = SL * NS
LEAK = 0.2

_mesh = functools.partial(
    plsc.VectorSubcoreMesh, core_axis_name="c", subcore_axis_name="s",
    num_cores=NC, num_subcores=NS)
_SC_PARAMS = pltpu.CompilerParams(use_tc_tiling_on_sc=False,
                                  needs_layout_passes=False)


def _zero_rows(ref, nrows):
    def body(i, _):
        ref[i, :] = jnp.zeros((16,), jnp.float32)
        return 0
    lax.fori_loop(0, nrows, body, 0)


def _zero_flat(ref, n):
    def body(i, _):
        ref[pl.ds(i * 16, 16)] = jnp.zeros((16,), jnp.float32)
        return 0
    lax.fori_loop(0, n // 16, body, 0)


def _k1_body(attr_hbm, dst_hbm, seg_out, cnt_out,
             attr_v, dst_v, ones_v, zb_v, seg_sh, cnt_sh):
    c = lax.axis_index("c")
    s = lax.axis_index("s")
    eb = (c * NS + s) * EPT
    nb = s * SL

    _zero_rows(attr_v, SL)
    _zero_flat(zb_v, SL)
    def ones_body(i, _):
        ones_v[pl.ds(i * 16, 16)] = jnp.ones((16,), jnp.float32)
        return 0
    lax.fori_loop(0, CHUNK // 16, ones_body, 0)
    pltpu.sync_copy(attr_v.at[pl.ds(0, SL)], seg_sh.at[pl.ds(nb, SL)])
    pltpu.sync_copy(zb_v, cnt_sh.at[pl.ds(nb, SL)])
    plsc.subcore_barrier()

    for j in range(NCHUNKS):
        off = eb + j * CHUNK
        pltpu.sync_copy(attr_hbm.at[pl.ds(off, CHUNK)], attr_v)
        pltpu.sync_copy(dst_hbm.at[pl.ds(off, CHUNK)], dst_v)
        pltpu.sync_copy(attr_v, seg_sh.at[dst_v], add=True)
        pltpu.sync_copy(ones_v, cnt_sh.at[dst_v], add=True)

    plsc.subcore_barrier()
    pltpu.sync_copy(seg_sh.at[pl.ds(nb, SL)], seg_out.at[c, pl.ds(nb, SL)])
    pltpu.sync_copy(cnt_sh.at[pl.ds(nb, SL)], cnt_out.at[c, pl.ds(nb, SL)])


def _k1(edge_attr, dst):
    return pl.kernel(
        _k1_body,
        out_type=(jax.ShapeDtypeStruct((NC, NPAD, ED), jnp.float32),
                  jax.ShapeDtypeStruct((NC, NPAD), jnp.float32)),
        mesh=_mesh(),
        compiler_params=_SC_PARAMS,
        scratch_types=[
            pltpu.VMEM((CHUNK, ED), jnp.float32),
            pltpu.VMEM((CHUNK,), jnp.int32),
            pltpu.VMEM((CHUNK,), jnp.float32),
            pltpu.VMEM((SL,), jnp.float32),
            pltpu.VMEM_SHARED((NPAD, ED), jnp.float32),
            pltpu.VMEM_SHARED((NPAD,), jnp.float32),
        ],
    )(edge_attr, dst)


def _k2_body(x_ref, seg_ref, cnt_ref, we_ref, be_ref, wg_ref, asr_ref,
             adr_ref, asrc_out, adst_out, pself_out):
    x = x_ref[...]
    seg = seg_ref[0] + seg_ref[1]
    cnt = cnt_ref[0] + cnt_ref[1]
    wg = wg_ref[...]
    we = we_ref[...]
    be = be_ref[...]

    v_src = jnp.sum(asr_ref[...][:, None] * wg, axis=0)
    v_dst = jnp.sum(adr_ref[...][:, None] * wg, axis=0)
    u_src = jnp.sum(v_src[:, None] * we, axis=0)
    u_dst = jnp.sum(v_dst[:, None] * we, axis=0)
    cb_src = jnp.sum(v_src * be)
    cb_dst = jnp.sum(v_dst * be)

    cmax = jnp.maximum(cnt, 1.0)
    a_src = (jnp.sum(x * v_src[None, :], axis=1)
             + (jnp.sum(seg * u_src[None, :], axis=1) + cnt * cb_src) / cmax)
    a_dst = (jnp.sum(x * v_dst[None, :], axis=1)
             + (jnp.sum(seg * u_dst[None, :], axis=1) + cnt * cb_dst) / cmax)
    sself = a_src + a_dst
    sself = jnp.where(sself >= 0, sself, LEAK * sself)
    asrc_out[...] = a_src
    adst_out[...] = a_dst
    pself_out[...] = jnp.exp(sself)


def _k2(x, seg, cnt, W_edge, b_edge, W_gat, att_src, att_dst):
    return pl.pallas_call(
        _k2_body,
        out_shape=(jax.ShapeDtypeStruct((N,), jnp.float32),
                   jax.ShapeDtypeStruct((N,), jnp.float32),
                   jax.ShapeDtypeStruct((N,), jnp.float32)),
    )(x, seg, cnt, W_edge, b_edge, W_gat, att_src, att_dst)


def _k3a_body(src_hbm, dst_hbm, asrc_hbm, adst_hbm, p_out, dn_out,
              si_v, di_v, p_v, as_v, ad_v, zb_v, dn_sh):
    c = lax.axis_index("c")
    s = lax.axis_index("s")
    eb = (c * NS + s) * EPT
    nb = s * SL

    pltpu.sync_copy(asrc_hbm, as_v)
    pltpu.sync_copy(adst_hbm, ad_v)
    _zero_flat(zb_v, SL)
    pltpu.sync_copy(zb_v, dn_sh.at[pl.ds(nb, SL)])
    plsc.subcore_barrier()

    for j in range(NCHUNKS):
        off = eb + j * CHUNK
        pltpu.sync_copy(src_hbm.at[pl.ds(off, CHUNK)], si_v)
        pltpu.sync_copy(dst_hbm.at[pl.ds(off, CHUNK)], di_v)

        def body(g, _):
            i0 = pl.multiple_of(g * 16, 16)
            a1 = plsc.load_gather(as_v, [si_v[pl.ds(i0, 16)]])
            a2 = plsc.load_gather(ad_v, [di_v[pl.ds(i0, 16)]])
            sv = a1 + a2
            sv = jnp.where(sv >= 0, sv, LEAK * sv)
            p_v[pl.ds(i0, 16)] = jnp.exp(sv)
            return 0
        lax.fori_loop(0, CHUNK // 16, body, 0)

        pltpu.sync_copy(p_v, p_out.at[pl.ds(off, CHUNK)])
        pltpu.sync_copy(p_v, dn_sh.at[di_v], add=True)

    plsc.subcore_barrier()
    pltpu.sync_copy(dn_sh.at[pl.ds(nb, SL)], dn_out.at[c, pl.ds(nb, SL)])


def _k3a(src, dst, a_src, a_dst):
    return pl.kernel(
        _k3a_body,
        out_type=(jax.ShapeDtypeStruct((E,), jnp.float32),
                  jax.ShapeDtypeStruct((NC, NPAD), jnp.float32)),
        mesh=_mesh(),
        compiler_params=_SC_PARAMS,
        scratch_types=[
            pltpu.VMEM((CHUNK,), jnp.int32),
            pltpu.VMEM((CHUNK,), jnp.int32),
            pltpu.VMEM((CHUNK,), jnp.float32),
            pltpu.VMEM((N,), jnp.float32),
            pltpu.VMEM((N,), jnp.float32),
            pltpu.VMEM((SL,), jnp.float32),
            pltpu.VMEM_SHARED((NPAD,), jnp.float32),
        ],
    )(src, dst, a_src, a_dst)


def _k3b_body(src_hbm, dst_hbm, p_hbm, dnp_hbm, pself_hbm, g_out,
              si_v, di_v, p_v, dn_v, t_v, zb_v, g_sh):
    c = lax.axis_index("c")
    s = lax.axis_index("s")
    eb = (c * NS + s) * EPT
    nb = s * SL

    pltpu.sync_copy(dnp_hbm.at[0, pl.ds(0, N)], dn_v)
    pltpu.sync_copy(dnp_hbm.at[1, pl.ds(0, N)], t_v)
    def addt(i, _):
        i0 = pl.multiple_of(i * 16, 16)
        dn_v[pl.ds(i0, 16)] = dn_v[pl.ds(i0, 16)] + t_v[pl.ds(i0, 16)]
        return 0
    lax.fori_loop(0, N // 16, addt, 0)
    pltpu.sync_copy(pself_hbm, t_v)
    def addp(i, _):
        i0 = pl.multiple_of(i * 16, 16)
        dn_v[pl.ds(i0, 16)] = (dn_v[pl.ds(i0, 16)] + t_v[pl.ds(i0, 16)]
                               + 1e-16)
        return 0
    lax.fori_loop(0, N // 16, addp, 0)

    _zero_flat(zb_v, SL)
    pltpu.sync_copy(zb_v, g_sh.at[pl.ds(nb, SL)])
    plsc.subcore_barrier()

    for j in range(NCHUNKS):
        off = eb + j * CHUNK
        pltpu.sync_copy(src_hbm.at[pl.ds(off, CHUNK)], si_v)
        pltpu.sync_copy(dst_hbm.at[pl.ds(off, CHUNK)], di_v)
        pltpu.sync_copy(p_hbm.at[pl.ds(off, CHUNK)], p_v)

        def body(g, _):
            i0 = pl.multiple_of(g * 16, 16)
            d = plsc.load_gather(dn_v, [di_v[pl.ds(i0, 16)]])
            p_v[pl.ds(i0, 16)] = p_v[pl.ds(i0, 16)] / d
            return 0
        lax.fori_loop(0, CHUNK // 16, body, 0)

        pltpu.sync_copy(p_v, g_sh.at[si_v], add=True)

    plsc.subcore_barrier()
    pltpu.sync_copy(g_sh.at[pl.ds(nb, SL)], g_out.at[c, pl.ds(nb, SL)])


def _k3b(src, dst, p, dn_part, pself):
    return pl.kernel(
        _k3b_body,
        out_type=jax.ShapeDtypeStruct((NC, NPAD), jnp.float32),
        mesh=_mesh(),
        compiler_params=_SC_PARAMS,
        scratch_types=[
            pltpu.VMEM((CHUNK,), jnp.int32),
            pltpu.VMEM((CHUNK,), jnp.int32),
            pltpu.VMEM((CHUNK,), jnp.float32),
            pltpu.VMEM((N,), jnp.float32),
            pltpu.VMEM((N,), jnp.float32),
            pltpu.VMEM((SL,), jnp.float32),
            pltpu.VMEM_SHARED((NPAD,), jnp.float32),
        ],
    )(src, dst, p, dn_part, pself)


def _k4_body(x_ref, seg_ref, cnt_ref, gp_ref, dnp_ref, pself_ref, we_ref,
             be_ref, wg_ref, bg_ref, out_ref):
    x = x_ref[...]
    seg = seg_ref[0] + seg_ref[1]
    cnt = cnt_ref[0] + cnt_ref[1]
    pself = pself_ref[...]
    den = dnp_ref[0] + dnp_ref[1] + pself + 1e-16
    g = gp_ref[0] + gp_ref[1] + pself / den
    cmax = jnp.maximum(cnt, 1.0)

    t128 = jnp.dot(g.reshape(1, N), x,
                   preferred_element_type=jnp.float32)
    t16 = jnp.dot((g / cmax).reshape(1, N), seg,
                  preferred_element_type=jnp.float32)
    scal = jnp.sum(g * cnt / cmax)
    total = (t128
             + lax.dot_general(t16, we_ref[...], (((1,), (1,)), ((), ())),
                               preferred_element_type=jnp.float32)
             + scal * be_ref[...][None, :])
    outv = lax.dot_general(total, wg_ref[...], (((1,), (1,)), ((), ())),
                           preferred_element_type=jnp.float32)
    out_ref[...] = outv.reshape(ND) * (1.0 / N) + bg_ref[...]


def _k4(x, seg, cnt, gp, dnp, pself, W_edge, b_edge, W_gat, bias_gat):
    return pl.pallas_call(
        _k4_body,
        out_shape=jax.ShapeDtypeStruct((ND,), jnp.float32),
    )(x, seg, cnt, gp, dnp, pself, W_edge, b_edge, W_gat, bias_gat)


def kernel(x, edge_index, edge_attr, W_edge, b_edge, W_gat, att_src,
           att_dst, bias_gat):
    src = edge_index[0].astype(jnp.int32)
    dst = edge_index[1].astype(jnp.int32)

    seg_p, cnt_p = _k1(edge_attr, dst)
    seg_p = seg_p[:, :N]
    cnt_p = cnt_p[:, :N]
    a_src, a_dst, pself = _k2(x, seg_p, cnt_p, W_edge, b_edge, W_gat,
                              att_src, att_dst)
    p, dn_p = _k3a(src, dst, a_src, a_dst)
    g_p = _k3b(src, dst, p, dn_p, pself)
    return _k4(x, seg_p, cnt_p, g_p[:, :N], dn_p[:, :N], pself,
               W_edge, b_edge, W_gat, bias_gat)


# speedup vs baseline: 41.9972x; 1.0256x over previous
import functools

import jax
import jax.numpy as jnp
from jax import lax
from jax.experimental import pallas as pl
from jax.experimental.pallas import tpu as pltpu
from jax.experimental.pallas import tpu_sc as plsc

N = 10000
E = 320000
ED = 16
ND = 128

NC = 2
NS = 16
NW = NC * NS
EPT = E // NW
CHUNK = 2000
NCHUNKS = EPT // CHUNK
SL = 640
NPAD = SL * NS
LEAK = 0.2

_mesh = functools.partial(
    plsc.VectorSubcoreMesh, core_axis_name="c", subcore_axis_name="s",
    num_cores=NC, num_subcores=NS)
_SC_PARAMS = pltpu.CompilerParams(use_tc_tiling_on_sc=False,
                                  needs_layout_passes=False)


def _zero_rows(ref, nrows):
    def body(i, _):
        ref[i, :] = jnp.zeros((16,), jnp.float32)
        return 0
    lax.fori_loop(0, nrows, body, 0)


def _zero_flat(ref, n):
    def body(i, _):
        ref[pl.ds(i * 16, 16)] = jnp.zeros((16,), jnp.float32)
        return 0
    lax.fori_loop(0, n // 16, body, 0)


def _k1_body(attr_hbm, ei_hbm, seg_out, cnt_out,
             attr_v, dst_v, ones_v, zb_v, seg_sh, cnt_sh):
    c = lax.axis_index("c")
    s = lax.axis_index("s")
    eb = (c * NS + s) * EPT
    nb = s * SL

    _zero_rows(attr_v, SL)
    _zero_flat(zb_v, SL)
    def ones_body(i, _):
        ones_v[pl.ds(i * 16, 16)] = jnp.ones((16,), jnp.float32)
        return 0
    lax.fori_loop(0, CHUNK // 16, ones_body, 0)
    pltpu.sync_copy(attr_v.at[pl.ds(0, SL)], seg_sh.at[pl.ds(nb, SL)])
    pltpu.sync_copy(zb_v, cnt_sh.at[pl.ds(nb, SL)])
    plsc.subcore_barrier()

    for j in range(NCHUNKS):
        off = eb + j * CHUNK
        pltpu.sync_copy(attr_hbm.at[pl.ds(off, CHUNK)], attr_v)
        pltpu.sync_copy(ei_hbm.at[1, pl.ds(off, CHUNK)], dst_v)
        pltpu.sync_copy(attr_v, seg_sh.at[dst_v], add=True)
        pltpu.sync_copy(ones_v, cnt_sh.at[dst_v], add=True)

    plsc.subcore_barrier()
    pltpu.sync_copy(seg_sh.at[pl.ds(nb, SL)], seg_out.at[c, pl.ds(nb, SL)])
    pltpu.sync_copy(cnt_sh.at[pl.ds(nb, SL)], cnt_out.at[c, pl.ds(nb, SL)])


def _k1(edge_attr, ei):
    return pl.kernel(
        _k1_body,
        out_type=(jax.ShapeDtypeStruct((NC, NPAD, ED), jnp.float32),
                  jax.ShapeDtypeStruct((NC, NPAD), jnp.float32)),
        mesh=_mesh(),
        compiler_params=_SC_PARAMS,
        scratch_types=[
            pltpu.VMEM((CHUNK, ED), jnp.float32),
            pltpu.VMEM((CHUNK,), jnp.int32),
            pltpu.VMEM((CHUNK,), jnp.float32),
            pltpu.VMEM((SL,), jnp.float32),
            pltpu.VMEM_SHARED((NPAD, ED), jnp.float32),
            pltpu.VMEM_SHARED((NPAD,), jnp.float32),
        ],
    )(edge_attr, ei)


def _k2_body(x_ref, seg_ref, cnt_ref, we_ref, be_ref, wg_ref, asr_ref,
             adr_ref, asrc_out, adst_out, pself_out):
    x = x_ref[...]
    seg = seg_ref[0, :N] + seg_ref[1, :N]
    cnt = cnt_ref[0, :N] + cnt_ref[1, :N]
    wg = wg_ref[...]
    we = we_ref[...]
    be = be_ref[...]

    v_src = jnp.sum(asr_ref[...][:, None] * wg, axis=0)
    v_dst = jnp.sum(adr_ref[...][:, None] * wg, axis=0)
    u_src = jnp.sum(v_src[:, None] * we, axis=0)
    u_dst = jnp.sum(v_dst[:, None] * we, axis=0)
    cb_src = jnp.sum(v_src * be)
    cb_dst = jnp.sum(v_dst * be)

    cmax = jnp.maximum(cnt, 1.0)
    a_src = (jnp.sum(x * v_src[None, :], axis=1)
             + (jnp.sum(seg * u_src[None, :], axis=1) + cnt * cb_src) / cmax)
    a_dst = (jnp.sum(x * v_dst[None, :], axis=1)
             + (jnp.sum(seg * u_dst[None, :], axis=1) + cnt * cb_dst) / cmax)
    sself = a_src + a_dst
    sself = jnp.where(sself >= 0, sself, LEAK * sself)
    asrc_out[...] = a_src
    adst_out[...] = a_dst
    pself_out[...] = jnp.exp(sself)


def _k2(x, seg, cnt, W_edge, b_edge, W_gat, att_src, att_dst):
    return pl.pallas_call(
        _k2_body,
        out_shape=(jax.ShapeDtypeStruct((N,), jnp.float32),
                   jax.ShapeDtypeStruct((N,), jnp.float32),
                   jax.ShapeDtypeStruct((N,), jnp.float32)),
    )(x, seg, cnt, W_edge, b_edge, W_gat, att_src, att_dst)


def _k3a_body(ei_hbm, asrc_hbm, adst_hbm, p_out, dn_out,
              si_v, di_v, p_v, as_v, ad_v, zb_v, dn_sh):
    c = lax.axis_index("c")
    s = lax.axis_index("s")
    eb = (c * NS + s) * EPT
    nb = s * SL

    pltpu.sync_copy(asrc_hbm, as_v)
    pltpu.sync_copy(adst_hbm, ad_v)
    _zero_flat(zb_v, SL)
    pltpu.sync_copy(zb_v, dn_sh.at[pl.ds(nb, SL)])
    plsc.subcore_barrier()

    for j in range(NCHUNKS):
        off = eb + j * CHUNK
        pltpu.sync_copy(ei_hbm.at[0, pl.ds(off, CHUNK)], si_v)
        pltpu.sync_copy(ei_hbm.at[1, pl.ds(off, CHUNK)], di_v)

        def body(g, _):
            i0 = pl.multiple_of(g * 16, 16)
            a1 = plsc.load_gather(as_v, [si_v[pl.ds(i0, 16)]])
            a2 = plsc.load_gather(ad_v, [di_v[pl.ds(i0, 16)]])
            sv = a1 + a2
            sv = jnp.where(sv >= 0, sv, LEAK * sv)
            p_v[pl.ds(i0, 16)] = jnp.exp(sv)
            return 0
        lax.fori_loop(0, CHUNK // 16, body, 0)

        pltpu.sync_copy(p_v, p_out.at[pl.ds(off, CHUNK)])
        pltpu.sync_copy(p_v, dn_sh.at[di_v], add=True)

    plsc.subcore_barrier()
    pltpu.sync_copy(dn_sh.at[pl.ds(nb, SL)], dn_out.at[c, pl.ds(nb, SL)])


def _k3a(ei, a_src, a_dst):
    return pl.kernel(
        _k3a_body,
        out_type=(jax.ShapeDtypeStruct((E,), jnp.float32),
                  jax.ShapeDtypeStruct((NC, NPAD), jnp.float32)),
        mesh=_mesh(),
        compiler_params=_SC_PARAMS,
        scratch_types=[
            pltpu.VMEM((CHUNK,), jnp.int32),
            pltpu.VMEM((CHUNK,), jnp.int32),
            pltpu.VMEM((CHUNK,), jnp.float32),
            pltpu.VMEM((N,), jnp.float32),
            pltpu.VMEM((N,), jnp.float32),
            pltpu.VMEM((SL,), jnp.float32),
            pltpu.VMEM_SHARED((NPAD,), jnp.float32),
        ],
    )(ei, a_src, a_dst)


def _k3b_body(ei_hbm, p_hbm, dnp_hbm, pself_hbm, g_out,
              si_v, di_v, p_v, dn_v, t_v, zb_v, g_sh):
    c = lax.axis_index("c")
    s = lax.axis_index("s")
    eb = (c * NS + s) * EPT
    nb = s * SL

    pltpu.sync_copy(dnp_hbm.at[0, pl.ds(0, N)], dn_v)
    pltpu.sync_copy(dnp_hbm.at[1, pl.ds(0, N)], t_v)
    def addt(i, _):
        i0 = pl.multiple_of(i * 16, 16)
        dn_v[pl.ds(i0, 16)] = dn_v[pl.ds(i0, 16)] + t_v[pl.ds(i0, 16)]
        return 0
    lax.fori_loop(0, N // 16, addt, 0)
    pltpu.sync_copy(pself_hbm, t_v)
    def addp(i, _):
        i0 = pl.multiple_of(i * 16, 16)
        dn_v[pl.ds(i0, 16)] = (dn_v[pl.ds(i0, 16)] + t_v[pl.ds(i0, 16)]
                               + 1e-16)
        return 0
    lax.fori_loop(0, N // 16, addp, 0)

    _zero_flat(zb_v, SL)
    pltpu.sync_copy(zb_v, g_sh.at[pl.ds(nb, SL)])
    plsc.subcore_barrier()

    for j in range(NCHUNKS):
        off = eb + j * CHUNK
        pltpu.sync_copy(ei_hbm.at[0, pl.ds(off, CHUNK)], si_v)
        pltpu.sync_copy(ei_hbm.at[1, pl.ds(off, CHUNK)], di_v)
        pltpu.sync_copy(p_hbm.at[pl.ds(off, CHUNK)], p_v)

        def body(g, _):
            i0 = pl.multiple_of(g * 16, 16)
            d = plsc.load_gather(dn_v, [di_v[pl.ds(i0, 16)]])
            p_v[pl.ds(i0, 16)] = p_v[pl.ds(i0, 16)] / d
            return 0
        lax.fori_loop(0, CHUNK // 16, body, 0)

        pltpu.sync_copy(p_v, g_sh.at[si_v], add=True)

    plsc.subcore_barrier()
    pltpu.sync_copy(g_sh.at[pl.ds(nb, SL)], g_out.at[c, pl.ds(nb, SL)])


def _k3b(ei, p, dn_part, pself):
    return pl.kernel(
        _k3b_body,
        out_type=jax.ShapeDtypeStruct((NC, NPAD), jnp.float32),
        mesh=_mesh(),
        compiler_params=_SC_PARAMS,
        scratch_types=[
            pltpu.VMEM((CHUNK,), jnp.int32),
            pltpu.VMEM((CHUNK,), jnp.int32),
            pltpu.VMEM((CHUNK,), jnp.float32),
            pltpu.VMEM((N,), jnp.float32),
            pltpu.VMEM((N,), jnp.float32),
            pltpu.VMEM((SL,), jnp.float32),
            pltpu.VMEM_SHARED((NPAD,), jnp.float32),
        ],
    )(ei, p, dn_part, pself)


def _k4_body(x_ref, seg_ref, cnt_ref, gp_ref, dnp_ref, pself_ref, we_ref,
             be_ref, wg_ref, bg_ref, out_ref):
    x = x_ref[...]
    seg = seg_ref[0, :N] + seg_ref[1, :N]
    cnt = cnt_ref[0, :N] + cnt_ref[1, :N]
    pself = pself_ref[...]
    den = dnp_ref[0, :N] + dnp_ref[1, :N] + pself + 1e-16
    g = gp_ref[0, :N] + gp_ref[1, :N] + pself / den
    cmax = jnp.maximum(cnt, 1.0)

    t128 = jnp.dot(g.reshape(1, N), x,
                   preferred_element_type=jnp.float32)
    t16 = jnp.dot((g / cmax).reshape(1, N), seg,
                  preferred_element_type=jnp.float32)
    scal = jnp.sum(g * cnt / cmax)
    total = (t128
             + lax.dot_general(t16, we_ref[...], (((1,), (1,)), ((), ())),
                               preferred_element_type=jnp.float32)
             + scal * be_ref[...][None, :])
    outv = lax.dot_general(total, wg_ref[...], (((1,), (1,)), ((), ())),
                           preferred_element_type=jnp.float32)
    out_ref[...] = outv.reshape(ND) * (1.0 / N) + bg_ref[...]


def _k4(x, seg, cnt, gp, dnp, pself, W_edge, b_edge, W_gat, bias_gat):
    return pl.pallas_call(
        _k4_body,
        out_shape=jax.ShapeDtypeStruct((ND,), jnp.float32),
    )(x, seg, cnt, gp, dnp, pself, W_edge, b_edge, W_gat, bias_gat)


def kernel(x, edge_index, edge_attr, W_edge, b_edge, W_gat, att_src,
           att_dst, bias_gat):
    ei = edge_index.astype(jnp.int32)

    seg_p, cnt_p = _k1(edge_attr, ei)
    a_src, a_dst, pself = _k2(x, seg_p, cnt_p, W_edge, b_edge, W_gat,
                              att_src, att_dst)
    p, dn_p = _k3a(ei, a_src, a_dst)
    g_p = _k3b(ei, p, dn_p, pself)
    return _k4(x, seg_p, cnt_p, g_p, dn_p, pself,
               W_edge, b_edge, W_gat, bias_gat)
